# baseline (speedup 1.0000x reference)
import functools

import jax
import jax.numpy as jnp
from jax import lax
from jax.experimental import pallas as pl
from jax.experimental.pallas import tpu as pltpu
from jax.experimental.pallas import tpu_sc as plsc

N_TOK = 16384
N_EXP = 64
K = 8
NC = 2
NS = 16
NW = NC * NS
TPW = N_TOK // NW
SPW = TPW * K
GRID = 8
BT = N_TOK // GRID
CPB = BT // TPW


def _tc_body(logits_ref, probs_ref, scores_ref, assign_ref, bhist_ref):
    x = logits_ref[:]
    m = jnp.max(x, axis=1, keepdims=True)
    e = jnp.exp(x - m)
    p = e / jnp.sum(e, axis=1, keepdims=True)
    probs_ref[:] = p
    lane_rev = (
        (N_EXP - 1) - lax.broadcasted_iota(jnp.int32, (BT, N_EXP), 1)
    ).astype(jnp.float32)
    work = p
    for k in range(K):
        mk = jnp.max(work, axis=1, keepdims=True)
        cand = jnp.where(work == mk, lane_rev, -1.0)
        mrev = jnp.max(cand, axis=1, keepdims=True)
        work = jnp.where(cand == mrev, -1.0, work)
        scores_ref[:, k:k + 1] = mk
        assign_ref[:, k:k + 1] = ((N_EXP - 1.0) - mrev).astype(jnp.int32)
    sel_all = (work < 0.0).astype(jnp.int32)
    for g in range(CPB):
        bhist_ref[g, 0, :] = jnp.sum(
            sel_all[g * TPW:(g + 1) * TPW], axis=0
        )


_tc_call = pl.pallas_call(
    _tc_body,
    grid=(GRID,),
    in_specs=[pl.BlockSpec((BT, N_EXP), lambda i: (i, 0))],
    out_specs=[
        pl.BlockSpec((BT, N_EXP), lambda i: (i, 0)),
        pl.BlockSpec((BT, K), lambda i: (i, 0)),
        pl.BlockSpec((BT, K), lambda i: (i, 0)),
        pl.BlockSpec((CPB, 1, N_EXP), lambda i: (i, 0, 0)),
    ],
    out_shape=[
        jax.ShapeDtypeStruct((N_TOK, N_EXP), jnp.float32),
        jax.ShapeDtypeStruct((N_TOK, K), jnp.float32),
        jax.ShapeDtypeStruct((N_TOK, K), jnp.int32),
        jax.ShapeDtypeStruct((NW, 1, N_EXP), jnp.int32),
    ],
)


def _sc_body(assign_hbm, bhist_hbm, counts_out, offs_out,
             bh_v, a_v, o_v, hist_v, tot_v):
    c = lax.axis_index("c")
    s = lax.axis_index("s")
    wid = s * NC + c
    pltpu.sync_copy(bhist_hbm, bh_v)
    pltpu.sync_copy(assign_hbm.at[pl.ds(wid * TPW, TPW), :], a_v)
    for j in range(N_EXP // 16):
        acc = jnp.zeros((16,), jnp.int32)
        tot = jnp.zeros((16,), jnp.int32)
        for u in range(NW):
            v = bh_v[pl.ds(u * N_EXP + j * 16, 16)]
            pre = (jnp.int32(u) < wid).astype(jnp.int32)
            acc = acc + v * pre
            tot = tot + v
        hist_v[pl.ds(j * 16, 16)] = acc
        tot_v[pl.ds(j * 16, 16)] = tot

    lane = lax.broadcasted_iota(jnp.int32, (16,), 0)
    mask8 = lane < K
    ones = jnp.ones((16,), jnp.int32)

    def tok_body(t, carry):
        trow = jnp.full((16,), t, jnp.int32)
        idx = plsc.load_gather(a_v, [trow, lane], mask=mask8)
        g = plsc.load_gather(hist_v, [idx], mask=mask8)
        plsc.addupdate_scatter(hist_v, [idx], ones, mask=mask8)
        plsc.store_scatter(o_v, [trow, lane], g, mask=mask8)
        return carry

    lax.fori_loop(0, TPW, tok_body, 0)
    pltpu.sync_copy(o_v, offs_out.at[pl.ds(wid * TPW, TPW), :])

    @pl.when(wid == 0)
    def _():
        pltpu.sync_copy(tot_v, counts_out)


@functools.cache
def _sc_call():
    mesh = plsc.VectorSubcoreMesh(
        core_axis_name="c", subcore_axis_name="s", num_cores=NC, num_subcores=NS
    )
    return functools.partial(
        pl.kernel,
        mesh=mesh,
        compiler_params=pltpu.CompilerParams(
            needs_layout_passes=False, use_tc_tiling_on_sc=False
        ),
        out_type=[
            jax.ShapeDtypeStruct((N_EXP,), jnp.int32),
            jax.ShapeDtypeStruct((N_TOK, K), jnp.int32),
        ],
        scratch_types=[
            pltpu.VMEM((NW * N_EXP,), jnp.int32),
            pltpu.VMEM((TPW, K), jnp.int32),
            pltpu.VMEM((TPW, K), jnp.int32),
            pltpu.VMEM((N_EXP,), jnp.int32),
            pltpu.VMEM((N_EXP,), jnp.int32),
        ],
    )(_sc_body)


def kernel(expert_counts, assignments, offsets, logits):
    probs, scores, assign, bhist = _tc_call(logits)
    counts, offs = _sc_call()(assign, bhist.reshape(-1))
    return counts, scores, assign, offs, probs


# speedup vs baseline: 2.4790x; 2.4790x over previous
import functools

import jax
import jax.numpy as jnp
from jax import lax
from jax.experimental import pallas as pl
from jax.experimental.pallas import tpu as pltpu
from jax.experimental.pallas import tpu_sc as plsc

N_TOK = 16384
N_EXP = 64
K = 8
NC = 2
NS = 16
NW = NC * NS
TPW = N_TOK // NW
SPW = TPW * K
GRID = 8
BT = N_TOK // GRID
CPB = BT // TPW
LANES = 128
JT = N_TOK // LANES


def _tc_body(xt_ref, probs_t_ref, scores_t_ref, assign_t_ref, asc_ref,
             bh_t_ref):
    x = xt_ref[:]
    m = jnp.max(x, axis=0, keepdims=True)
    e = jnp.exp(x - m)
    p = e / jnp.sum(e, axis=0, keepdims=True)
    probs_t_ref[:] = p
    sub_rev = (
        (N_EXP - 1) - lax.broadcasted_iota(jnp.int32, (N_EXP, BT), 0)
    ).astype(jnp.float32)
    work = p
    rows = []
    for k in range(K):
        mk = jnp.max(work, axis=0, keepdims=True)
        cand = jnp.where(work == mk, sub_rev, -1.0)
        mrev = jnp.max(cand, axis=0, keepdims=True)
        work = jnp.where(cand == mrev, -1.0, work)
        idx = ((N_EXP - 1.0) - mrev).astype(jnp.int32)
        rows.append(idx)
        scores_t_ref[k:k + 1, :] = mk
        assign_t_ref[k:k + 1, :] = idx
    at = jnp.concatenate(rows, axis=0)
    asc_ref[:] = at.reshape(K, BT // LANES, LANES).transpose(1, 0, 2).reshape(
        BT // LANES * K, LANES
    )
    sel = (work < 0.0).astype(jnp.int32)
    for g in range(CPB):
        bh_t_ref[0, :, g:g + 1] = jnp.sum(
            sel[:, g * TPW:(g + 1) * TPW], axis=1, keepdims=True
        )


_tc_call = pl.pallas_call(
    _tc_body,
    grid=(GRID,),
    in_specs=[pl.BlockSpec((N_EXP, BT), lambda i: (0, i))],
    out_specs=[
        pl.BlockSpec((N_EXP, BT), lambda i: (0, i)),
        pl.BlockSpec((K, BT), lambda i: (0, i)),
        pl.BlockSpec((K, BT), lambda i: (0, i)),
        pl.BlockSpec((BT // LANES * K, LANES), lambda i: (i, 0)),
        pl.BlockSpec((1, N_EXP, CPB), lambda i: (i, 0, 0)),
    ],
    out_shape=[
        jax.ShapeDtypeStruct((N_EXP, N_TOK), jnp.float32),
        jax.ShapeDtypeStruct((K, N_TOK), jnp.float32),
        jax.ShapeDtypeStruct((K, N_TOK), jnp.int32),
        jax.ShapeDtypeStruct((JT * K, LANES), jnp.int32),
        jax.ShapeDtypeStruct((GRID, N_EXP, CPB), jnp.int32),
    ],
)


def _sc_body(asc_hbm, bhist_hbm, counts_out, offs_sc_out,
             bh_v, a_v, o_v, hist_v, tot_v):
    c = lax.axis_index("c")
    s = lax.axis_index("s")
    wid = s * NC + c
    pltpu.sync_copy(bhist_hbm, bh_v)
    pltpu.sync_copy(asc_hbm.at[pl.ds(wid * SPW, SPW)], a_v)
    for j in range(N_EXP // 16):
        acc = jnp.zeros((16,), jnp.int32)
        tot = jnp.zeros((16,), jnp.int32)
        for u in range(NW):
            v = bh_v[pl.ds(u * N_EXP + j * 16, 16)]
            pre = (jnp.int32(u) < wid).astype(jnp.int32)
            acc = acc + v * pre
            tot = tot + v
        hist_v[pl.ds(j * 16, 16)] = acc
        tot_v[pl.ds(j * 16, 16)] = tot

    lane = lax.broadcasted_iota(jnp.int32, (16,), 0)
    mask8 = lane < K
    ones = jnp.ones((16,), jnp.int32)
    lane128 = lane * LANES

    def tok_body(tl, carry):
        base = ((tl >> 7) << 10) + (tl & (LANES - 1))
        slot = lane128 + base
        idx = plsc.load_gather(a_v, [slot], mask=mask8)
        g = plsc.load_gather(hist_v, [idx], mask=mask8)
        plsc.addupdate_scatter(hist_v, [idx], ones, mask=mask8)
        plsc.store_scatter(o_v, [slot], g, mask=mask8)
        return carry

    lax.fori_loop(0, TPW, tok_body, 0)
    pltpu.sync_copy(o_v, offs_sc_out.at[pl.ds(wid * SPW, SPW)])

    @pl.when(wid == 0)
    def _():
        pltpu.sync_copy(tot_v, counts_out)


@functools.cache
def _sc_call():
    mesh = plsc.VectorSubcoreMesh(
        core_axis_name="c", subcore_axis_name="s", num_cores=NC, num_subcores=NS
    )
    return functools.partial(
        pl.kernel,
        mesh=mesh,
        compiler_params=pltpu.CompilerParams(needs_layout_passes=False),
        out_type=[
            jax.ShapeDtypeStruct((N_EXP,), jnp.int32),
            jax.ShapeDtypeStruct((N_TOK * K,), jnp.int32),
        ],
        scratch_types=[
            pltpu.VMEM((N_EXP * NW,), jnp.int32),
            pltpu.VMEM((SPW,), jnp.int32),
            pltpu.VMEM((SPW,), jnp.int32),
            pltpu.VMEM((N_EXP,), jnp.int32),
            pltpu.VMEM((N_EXP,), jnp.int32),
        ],
    )(_sc_body)


def kernel(expert_counts, assignments, offsets, logits):
    probs_t, scores_t, assign_t, asc, bh_t = _tc_call(logits.T)
    bh_flat = bh_t.transpose(0, 2, 1).reshape(-1)
    counts, offs_flat = _sc_call()(asc.reshape(-1), bh_flat)
    offs = (
        offs_flat.reshape(JT, K, LANES)
        .transpose(0, 2, 1)
        .reshape(N_TOK, K)
    )
    return counts, scores_t.T, assign_t.T, offs, probs_t.T


# speedup vs baseline: 2.5081x; 1.0118x over previous
import functools

import jax
import jax.numpy as jnp
from jax import lax
from jax.experimental import pallas as pl
from jax.experimental.pallas import tpu as pltpu
from jax.experimental.pallas import tpu_sc as plsc

N_TOK = 16384
N_EXP = 64
K = 8
NC = 2
NS = 16
NW = NC * NS
TPW = N_TOK // NW
SPW = TPW * K
GRID = 8
BT = N_TOK // GRID
CPB = BT // TPW
LANES = 128
JT = N_TOK // LANES


def _tc_body(xt_ref, probs_t_ref, scores_t_ref, assign_t_ref, bh_t_ref):
    x = xt_ref[:]
    m = jnp.max(x, axis=0, keepdims=True)
    e = jnp.exp(x - m)
    p = e / jnp.sum(e, axis=0, keepdims=True)
    probs_t_ref[:] = p
    sub_rev = (
        (N_EXP - 1) - lax.broadcasted_iota(jnp.int32, (N_EXP, BT), 0)
    ).astype(jnp.float32)
    work = p
    rows = []
    for k in range(K):
        mk = jnp.max(work, axis=0, keepdims=True)
        cand = jnp.where(work == mk, sub_rev, -1.0)
        mrev = jnp.max(cand, axis=0, keepdims=True)
        work = jnp.where(cand == mrev, -1.0, work)
        idx = ((N_EXP - 1.0) - mrev).astype(jnp.int32)
        scores_t_ref[k:k + 1, :] = mk
        assign_t_ref[k:k + 1, :] = idx
    sel = (work < 0.0).astype(jnp.int32)
    for g in range(CPB):
        bh_t_ref[0, :, g:g + 1] = jnp.sum(
            sel[:, g * TPW:(g + 1) * TPW], axis=1, keepdims=True
        )


_tc_call = pl.pallas_call(
    _tc_body,
    grid=(GRID,),
    in_specs=[pl.BlockSpec((N_EXP, BT), lambda i: (0, i))],
    out_specs=[
        pl.BlockSpec((N_EXP, BT), lambda i: (0, i)),
        pl.BlockSpec((K, BT), lambda i: (0, i)),
        pl.BlockSpec((K, BT), lambda i: (0, i)),
        pl.BlockSpec((1, N_EXP, CPB), lambda i: (i, 0, 0)),
    ],
    out_shape=[
        jax.ShapeDtypeStruct((N_EXP, N_TOK), jnp.float32),
        jax.ShapeDtypeStruct((K, N_TOK), jnp.float32),
        jax.ShapeDtypeStruct((K, N_TOK), jnp.int32),
        jax.ShapeDtypeStruct((GRID, N_EXP, CPB), jnp.int32),
    ],
)


def _sc_body(asc_hbm, bhist_hbm, counts_out, offs_sc_out,
             bh_v, a_v, o_v, hist_v, tot_v):
    c = lax.axis_index("c")
    s = lax.axis_index("s")
    wid = s * NC + c
    pltpu.sync_copy(bhist_hbm, bh_v)
    pltpu.sync_copy(asc_hbm.at[pl.ds(wid * SPW, SPW)], a_v)
    for j in range(N_EXP // 16):
        acc = jnp.zeros((16,), jnp.int32)
        tot = jnp.zeros((16,), jnp.int32)
        for u in range(NW):
            v = bh_v[pl.ds(u * N_EXP + j * 16, 16)]
            pre = (jnp.int32(u) < wid).astype(jnp.int32)
            acc = acc + v * pre
            tot = tot + v
        hist_v[pl.ds(j * 16, 16)] = acc
        tot_v[pl.ds(j * 16, 16)] = tot

    lane = lax.broadcasted_iota(jnp.int32, (16,), 0)
    mask8 = lane < K
    ones = jnp.ones((16,), jnp.int32)
    lane128 = lane * LANES

    def tok_body(tl, carry):
        base = ((tl >> 7) << 10) + (tl & (LANES - 1))
        slot = lane128 + base
        idx = plsc.load_gather(a_v, [slot], mask=mask8)
        g = plsc.load_gather(hist_v, [idx], mask=mask8)
        plsc.addupdate_scatter(hist_v, [idx], ones, mask=mask8)
        plsc.store_scatter(o_v, [slot], g, mask=mask8)
        return carry

    lax.fori_loop(0, TPW, tok_body, 0)
    pltpu.sync_copy(o_v, offs_sc_out.at[pl.ds(wid * SPW, SPW)])

    @pl.when(wid == 0)
    def _():
        pltpu.sync_copy(tot_v, counts_out)


@functools.cache
def _sc_call():
    mesh = plsc.VectorSubcoreMesh(
        core_axis_name="c", subcore_axis_name="s", num_cores=NC, num_subcores=NS
    )
    return functools.partial(
        pl.kernel,
        mesh=mesh,
        compiler_params=pltpu.CompilerParams(needs_layout_passes=False),
        out_type=[
            jax.ShapeDtypeStruct((N_EXP,), jnp.int32),
            jax.ShapeDtypeStruct((N_TOK * K,), jnp.int32),
        ],
        scratch_types=[
            pltpu.VMEM((N_EXP * NW,), jnp.int32),
            pltpu.VMEM((SPW,), jnp.int32),
            pltpu.VMEM((SPW,), jnp.int32),
            pltpu.VMEM((N_EXP,), jnp.int32),
            pltpu.VMEM((N_EXP,), jnp.int32),
        ],
    )(_sc_body)


def kernel(expert_counts, assignments, offsets, logits):
    probs_t, scores_t, assign_t, bh_t = _tc_call(logits.T)
    asc = (
        assign_t.reshape(K, JT, LANES)
        .transpose(1, 0, 2)
        .reshape(-1)
    )
    bh_flat = bh_t.transpose(0, 2, 1).reshape(-1)
    counts, offs_flat = _sc_call()(asc, bh_flat)
    offs = (
        offs_flat.reshape(JT, K, LANES)
        .transpose(0, 2, 1)
        .reshape(N_TOK, K)
    )
    return counts, scores_t.T, assign_t.T, offs, probs_t.T


# speedup vs baseline: 2.6566x; 1.0592x over previous
import functools

import jax
import jax.numpy as jnp
from jax import lax
from jax.experimental import pallas as pl
from jax.experimental.pallas import tpu as pltpu
from jax.experimental.pallas import tpu_sc as plsc

N_TOK = 16384
N_EXP = 64
K = 8
NC = 2
NS = 16
NW = NC * NS
TPW = N_TOK // NW
SPW = TPW * K
GRID = 8
BT = N_TOK // GRID
CPB = BT // TPW
LANES = 128
JT = N_TOK // LANES


def _tc_body(xt_ref, probs_t_ref, scores_t_ref, assign_t_ref, bh_t_ref):
    x = xt_ref[:]
    m = jnp.max(x, axis=0, keepdims=True)
    e = jnp.exp(x - m)
    p = e / jnp.sum(e, axis=0, keepdims=True)
    probs_t_ref[:] = p
    sub_rev = (
        (N_EXP - 1) - lax.broadcasted_iota(jnp.int32, (N_EXP, BT), 0)
    ).astype(jnp.float32)
    work = p
    rows = []
    for k in range(K):
        mk = jnp.max(work, axis=0, keepdims=True)
        cand = jnp.where(work == mk, sub_rev, -1.0)
        mrev = jnp.max(cand, axis=0, keepdims=True)
        work = jnp.where(cand == mrev, -1.0, work)
        idx = ((N_EXP - 1.0) - mrev).astype(jnp.int32)
        scores_t_ref[k:k + 1, :] = mk
        assign_t_ref[k:k + 1, :] = idx
    sel = (work < 0.0).astype(jnp.int32)
    for g in range(CPB):
        bh_t_ref[0, :, g:g + 1] = jnp.sum(
            sel[:, g * TPW:(g + 1) * TPW], axis=1, keepdims=True
        )


_tc_call = pl.pallas_call(
    _tc_body,
    grid=(GRID,),
    in_specs=[pl.BlockSpec((N_EXP, BT), lambda i: (0, i))],
    out_specs=[
        pl.BlockSpec((N_EXP, BT), lambda i: (0, i)),
        pl.BlockSpec((K, BT), lambda i: (0, i)),
        pl.BlockSpec((K, BT), lambda i: (0, i)),
        pl.BlockSpec((1, N_EXP, CPB), lambda i: (i, 0, 0)),
    ],
    out_shape=[
        jax.ShapeDtypeStruct((N_EXP, N_TOK), jnp.float32),
        jax.ShapeDtypeStruct((K, N_TOK), jnp.float32),
        jax.ShapeDtypeStruct((K, N_TOK), jnp.int32),
        jax.ShapeDtypeStruct((GRID, N_EXP, CPB), jnp.int32),
    ],
)


def _sc_body(asc_hbm, bhist_hbm, counts_out, offs_sc_out,
             bh_v, a_v, o_v, hist_v, tot_v):
    c = lax.axis_index("c")
    s = lax.axis_index("s")
    wid = s * NC + c
    pltpu.sync_copy(bhist_hbm, bh_v)
    pltpu.sync_copy(asc_hbm.at[pl.ds(wid * SPW, SPW)], a_v)
    for j in range(N_EXP // 16):
        acc = jnp.zeros((16,), jnp.int32)
        tot = jnp.zeros((16,), jnp.int32)
        for u in range(NW):
            v = bh_v[pl.ds(u * N_EXP + j * 16, 16)]
            pre = (jnp.int32(u) < wid).astype(jnp.int32)
            acc = acc + v * pre
            tot = tot + v
        hist_v[pl.ds(j * 16, 16)] = acc
        tot_v[pl.ds(j * 16, 16)] = tot

    lane = lax.broadcasted_iota(jnp.int32, (16,), 0)
    ones = jnp.ones((16,), jnp.int32)
    hi = lane >> 3
    lo_mask = hi == 0
    hi_mask = hi == 1
    lane128 = (lane & (K - 1)) * LANES
    dnums = lax.GatherDimensionNumbers(
        offset_dims=(), collapsed_slice_dims=(0,), start_index_map=(0,)
    )

    def bcast_lane(v, j):
        starts = jnp.full((16, 1), j, jnp.int32)
        return lax.gather(
            v, starts, dnums, (1,),
            mode=lax.GatherScatterMode.PROMISE_IN_BOUNDS,
        )

    def tok_body(i, carry):
        tl = i * 2 + hi
        slot = ((tl >> 7) << 10) + (tl & (LANES - 1)) + lane128
        idx = plsc.load_gather(a_v, [slot])
        g = plsc.load_gather(hist_v, [idx])
        corr = jnp.zeros((16,), jnp.int32)
        for jx in range(K):
            corr = corr + (idx == bcast_lane(idx, jx)).astype(jnp.int32)
        g = g + jnp.where(hi_mask, corr, 0)
        plsc.store_scatter(o_v, [slot], g)
        plsc.addupdate_scatter(hist_v, [idx], ones, mask=lo_mask)
        plsc.addupdate_scatter(hist_v, [idx], ones, mask=hi_mask)
        return carry

    lax.fori_loop(0, TPW // 2, tok_body, 0)
    pltpu.sync_copy(o_v, offs_sc_out.at[pl.ds(wid * SPW, SPW)])

    @pl.when(wid == 0)
    def _():
        pltpu.sync_copy(tot_v, counts_out)


@functools.cache
def _sc_call():
    mesh = plsc.VectorSubcoreMesh(
        core_axis_name="c", subcore_axis_name="s", num_cores=NC, num_subcores=NS
    )
    return functools.partial(
        pl.kernel,
        mesh=mesh,
        compiler_params=pltpu.CompilerParams(needs_layout_passes=False),
        out_type=[
            jax.ShapeDtypeStruct((N_EXP,), jnp.int32),
            jax.ShapeDtypeStruct((N_TOK * K,), jnp.int32),
        ],
        scratch_types=[
            pltpu.VMEM((N_EXP * NW,), jnp.int32),
            pltpu.VMEM((SPW,), jnp.int32),
            pltpu.VMEM((SPW,), jnp.int32),
            pltpu.VMEM((N_EXP,), jnp.int32),
            pltpu.VMEM((N_EXP,), jnp.int32),
        ],
    )(_sc_body)


def kernel(expert_counts, assignments, offsets, logits):
    probs_t, scores_t, assign_t, bh_t = _tc_call(logits.T)
    asc = (
        assign_t.reshape(K, JT, LANES)
        .transpose(1, 0, 2)
        .reshape(-1)
    )
    bh_flat = bh_t.transpose(0, 2, 1).reshape(-1)
    counts, offs_flat = _sc_call()(asc, bh_flat)
    offs = (
        offs_flat.reshape(JT, K, LANES)
        .transpose(0, 2, 1)
        .reshape(N_TOK, K)
    )
    return counts, scores_t.T, assign_t.T, offs, probs_t.T
